# baseline (speedup 1.0000x reference)
import functools

import jax
import jax.numpy as jnp
from jax import lax
from jax.experimental import pallas as pl
from jax.experimental.pallas import tpu as pltpu
from jax.experimental.pallas import tpu_sc as plsc

_B, _Q, _C, _L = 16, 20000, 91, 20
_CP = 128
_LP = 32
_NC = 2
_CT = 59
_NSC = _C - _CT
_NU = 2 * _NSC
_CHUNKS = ((0, 7680), (7680, 7680), (15360, 4608))
_QTAIL = 19968


def _tc_reduce_body(x_ref, vals_ref, idx_ref):
    p = jax.nn.sigmoid(x_ref[0])
    bm = jnp.max(p, axis=1, keepdims=True)
    qio = lax.broadcasted_iota(jnp.int32, (_B, _Q), 1)
    bidx = jnp.min(jnp.where(p == bm, qio, _Q), axis=1, keepdims=True)
    vals_ref[0] = bm
    idx_ref[0] = bidx


def _tc_reduce(logits_t):
    return pl.pallas_call(
        _tc_reduce_body,
        grid=(_CT,),
        in_specs=[pl.BlockSpec((1, _B, _Q), lambda c: (c, 0, 0))],
        out_specs=[
            pl.BlockSpec((1, _B, 1), lambda c: (c, 0, 0)),
            pl.BlockSpec((1, _B, 1), lambda c: (c, 0, 0)),
        ],
        out_shape=[
            jax.ShapeDtypeStruct((_CT, _B, 1), jnp.float32),
            jax.ShapeDtypeStruct((_CT, _B, 1), jnp.int32),
        ],
    )(logits_t)


def _tc_tail_body(x_ref, vals_ref, idx_ref):
    lanes = lax.broadcasted_iota(jnp.int32, (_C, _B, 128), 2)
    valid = lanes < (_Q - _QTAIL)
    p = jnp.where(valid, jax.nn.sigmoid(x_ref[...]), -1.0)
    bm = jnp.max(p, axis=2, keepdims=True)
    bidx = jnp.min(jnp.where(p == bm, lanes + _QTAIL, _Q),
                   axis=2, keepdims=True)
    vals_ref[...] = bm
    idx_ref[...] = bidx


def _tc_tail(logits_t):
    return pl.pallas_call(
        _tc_tail_body,
        grid=(1,),
        in_specs=[pl.BlockSpec((_C, _B, 128), lambda i: (0, 0, _QTAIL // 128))],
        out_specs=[
            pl.BlockSpec((_C, _B, 1), lambda i: (0, 0, 0)),
            pl.BlockSpec((_C, _B, 1), lambda i: (0, 0, 0)),
        ],
        out_shape=[
            jax.ShapeDtypeStruct((_C, _B, 1), jnp.float32),
            jax.ShapeDtypeStruct((_C, _B, 1), jnp.int32),
        ],
    )(logits_t)


def _sc_reduce_body(x_hbm, vals_out, idx_out,
                    buf0, buf1, mstate, istate, vrow, irow, sem0, sem1):
    wid = lax.axis_index("s") * _NC + lax.axis_index("c")
    bufs = (buf0, buf1)
    sems = (sem0, sem1)
    lane = lax.iota(jnp.int32, 16)

    for rep in range(_NU // 32):
        u = wid + 32 * rep
        cc = _CT + u // 2
        tr = u % 2

        def _src(k):
            q0, w = _CHUNKS[k]
            return x_hbm.at[cc, pl.ds(8 * tr, 8), pl.ds(q0, w)]

        def _dst(k):
            w = _CHUNKS[k][1]
            buf = bufs[k % 2]
            return buf if w == 7680 else buf.at[:, pl.ds(0, w)]

        for b_loc in range(8):
            mstate[b_loc] = jnp.full((16,), -1.0, jnp.float32)
            istate[b_loc] = jnp.zeros((16,), jnp.int32)

        handles = [None, None, None]
        handles[0] = pltpu.async_copy(_src(0), _dst(0), sems[0])
        for k in range(3):
            if k + 1 < 3:
                handles[k + 1] = pltpu.async_copy(_src(k + 1), _dst(k + 1),
                                                  sems[(k + 1) % 2])
            handles[k].wait()
            q0, w = _CHUNKS[k]
            buf = bufs[k % 2]
            for b_loc in range(8):

                def _step(i, carry, b_loc=b_loc, buf=buf, q0=q0):
                    m, idx = carry
                    for j in range(4):
                        v = buf[b_loc, pl.ds(i * 64 + j * 16, 16)]
                        p = 1.0 / (1.0 + jnp.exp(-v))
                        qvec = lane + (q0 + i * 64 + j * 16)
                        gt = p > m
                        m = jnp.where(gt, p, m)
                        idx = jnp.where(gt, qvec, idx)
                    return (m, idx)

                m0 = mstate[b_loc]
                i0 = istate[b_loc]
                m, idx = lax.fori_loop(0, w // 64, _step, (m0, i0))
                mstate[b_loc] = m
                istate[b_loc] = idx

        for b_loc in range(8):
            m = mstate[b_loc]
            idx = istate[b_loc]
            gmax = jnp.max(m)
            gidx = jnp.min(jnp.where(m == gmax, idx, _Q))
            vrow[pl.ds(16 * b_loc, 16)] = jnp.full((16,), gmax, jnp.float32)
            irow[pl.ds(16 * b_loc, 16)] = jnp.full((16,), gidx, jnp.int32)
        pltpu.sync_copy(vrow, vals_out.at[u])
        pltpu.sync_copy(irow, idx_out.at[u])


@functools.cache
def _sc_reduce():
    return functools.partial(
        pl.kernel,
        mesh=plsc.VectorSubcoreMesh(core_axis_name="c", subcore_axis_name="s"),
        compiler_params=pltpu.CompilerParams(use_tc_tiling_on_sc=True,
                                             needs_layout_passes=False),
        out_type=[
            jax.ShapeDtypeStruct((_NU, 128), jnp.float32),
            jax.ShapeDtypeStruct((_NU, 128), jnp.int32),
        ],
        scratch_types=[
            pltpu.VMEM((8, 7680), jnp.float32),
            pltpu.VMEM((8, 7680), jnp.float32),
            pltpu.VMEM((8, 16), jnp.float32),
            pltpu.VMEM((8, 16), jnp.int32),
            pltpu.VMEM((128,), jnp.float32),
            pltpu.VMEM((128,), jnp.int32),
            pltpu.SemaphoreType.DMA,
            pltpu.SemaphoreType.DMA,
        ],
    )(_sc_reduce_body)


def _sc_gather_body(tcv_hbm, tci_hbm, tlv_hbm, tli_hbm, scv_hbm, sci_hbm,
                    lab_hbm, boxes_hbm,
                    scores_out, boxes_out,
                    lab_v, a1_v, a2_v, a3_v,
                    s_tcv, s_tci, s_tlv, s_tli, s_scv, s_sci,
                    sc_v, bidx_v, brow_v, sem):
    wid = lax.axis_index("s") * _NC + lax.axis_index("c")
    b = wid // 2
    pltpu.sync_copy(lab_hbm.at[wid], lab_v)
    labs = lab_v[...]
    a1_v[...] = jnp.minimum(labs, _CT - 1) * 16 + b
    a2_v[...] = labs * 16 + b
    a3_v[...] = (jnp.maximum(labs - _CT, 0) * 2 + b // 8) * 128 + (b % 8) * 16
    h = [
        pltpu.async_copy(tcv_hbm.at[a1_v], s_tcv, sem),
        pltpu.async_copy(tci_hbm.at[a1_v], s_tci, sem),
        pltpu.async_copy(tlv_hbm.at[a2_v], s_tlv, sem),
        pltpu.async_copy(tli_hbm.at[a2_v], s_tli, sem),
        pltpu.async_copy(scv_hbm.at[a3_v], s_scv, sem),
        pltpu.async_copy(sci_hbm.at[a3_v], s_sci, sem),
    ]
    for hh in h:
        hh.wait()
    g_tlv = s_tlv[...]
    g_scv = s_scv[...]
    later = g_tlv > g_scv
    v_s = jnp.where(later, g_tlv, g_scv)
    i_s = jnp.where(later, s_tli[...], s_sci[...])
    is_tc = labs < _CT
    sc_v[...] = jnp.where(is_tc, s_tcv[...], v_s)
    pltpu.sync_copy(sc_v, scores_out.at[wid])
    bidx_v[...] = jnp.where(is_tc, s_tci[...], i_s) + b * (4 * _Q)
    for k in range(4):
        a1_v[...] = bidx_v[...] + k * _Q
        pltpu.async_copy(boxes_hbm.at[a1_v], brow_v.at[k], sem).wait()
    pltpu.sync_copy(brow_v, boxes_out.at[wid])


@functools.cache
def _sc_gather():
    return functools.partial(
        pl.kernel,
        mesh=plsc.VectorSubcoreMesh(core_axis_name="c", subcore_axis_name="s"),
        compiler_params=pltpu.CompilerParams(use_tc_tiling_on_sc=False),
        out_type=[
            jax.ShapeDtypeStruct((_B * 2, 16), jnp.float32),
            jax.ShapeDtypeStruct((_B * 2, 4, 16), jnp.float32),
        ],
        scratch_types=[
            pltpu.VMEM((16,), jnp.int32),
            pltpu.VMEM((16,), jnp.int32),
            pltpu.VMEM((16,), jnp.int32),
            pltpu.VMEM((16,), jnp.int32),
            pltpu.VMEM((16,), jnp.float32),
            pltpu.VMEM((16,), jnp.int32),
            pltpu.VMEM((16,), jnp.float32),
            pltpu.VMEM((16,), jnp.int32),
            pltpu.VMEM((16,), jnp.float32),
            pltpu.VMEM((16,), jnp.int32),
            pltpu.VMEM((16,), jnp.float32),
            pltpu.VMEM((16,), jnp.int32),
            pltpu.VMEM((4, 16), jnp.float32),
            pltpu.SemaphoreType.DMA,
        ],
    )(_sc_gather_body)


def kernel(pred_logits, pred_boxes, target_sizes, target_labels):
    del target_sizes
    logits_t = pred_logits.transpose(2, 0, 1)
    vals_tc, idx_tc = _tc_reduce(logits_t)
    vals_tl, idx_tl = _tc_tail(logits_t)
    vals_sc, idx_sc = _sc_reduce()(logits_t)
    lab = jnp.pad(target_labels, ((0, 0), (0, _LP - _L))).reshape(_B * 2, 16)
    boxes_kq = pred_boxes.transpose(0, 2, 1).reshape(-1)
    scores32, boxes32 = _sc_gather()(
        vals_tc.reshape(-1), idx_tc.reshape(-1),
        vals_tl.reshape(-1), idx_tl.reshape(-1),
        vals_sc.reshape(-1), idx_sc.reshape(-1),
        lab, boxes_kq)
    scores = scores32.reshape(_B, _LP)[:, :_L]
    boxes = (boxes32.reshape(_B, 2, 4, 16).transpose(0, 1, 3, 2)
             .reshape(_B, _LP, 4)[:, :_L, :])
    return (scores, target_labels, boxes)


# speedup vs baseline: 1.0108x; 1.0108x over previous
import functools

import jax
import jax.numpy as jnp
from jax import lax
from jax.experimental import pallas as pl
from jax.experimental.pallas import tpu as pltpu
from jax.experimental.pallas import tpu_sc as plsc

_B, _Q, _C, _L = 16, 20000, 91, 20
_CP = 128
_LP = 32
_NC = 2
_CT = 59
_NSC = _C - _CT
_NU = 2 * _NSC
_CHUNKS = ((0, 7680), (7680, 7680), (15360, 4608))
_QTAIL = 19968


def _tc_reduce_body(x_ref, vals_ref, idx_ref):
    p = jax.nn.sigmoid(x_ref[0])
    bm = jnp.max(p, axis=1, keepdims=True)
    qio = lax.broadcasted_iota(jnp.int32, (_B, _Q), 1)
    bidx = jnp.min(jnp.where(p == bm, qio, _Q), axis=1, keepdims=True)
    vals_ref[0] = bm
    idx_ref[0] = bidx


def _tc_reduce(logits_t):
    return pl.pallas_call(
        _tc_reduce_body,
        grid=(_CT,),
        in_specs=[pl.BlockSpec((1, _B, _Q), lambda c: (c, 0, 0))],
        out_specs=[
            pl.BlockSpec((1, _B, 1), lambda c: (c, 0, 0)),
            pl.BlockSpec((1, _B, 1), lambda c: (c, 0, 0)),
        ],
        out_shape=[
            jax.ShapeDtypeStruct((_CT, _B, 1), jnp.float32),
            jax.ShapeDtypeStruct((_CT, _B, 1), jnp.int32),
        ],
    )(logits_t)


def _tc_tail_body(x_ref, vals_ref, idx_ref):
    lanes = lax.broadcasted_iota(jnp.int32, (_C, _B, 128), 2)
    valid = lanes < (_Q - _QTAIL)
    p = jnp.where(valid, jax.nn.sigmoid(x_ref[...]), -1.0)
    bm = jnp.max(p, axis=2, keepdims=True)
    bidx = jnp.min(jnp.where(p == bm, lanes + _QTAIL, _Q),
                   axis=2, keepdims=True)
    vals_ref[...] = bm
    idx_ref[...] = bidx


def _tc_tail(logits_t):
    return pl.pallas_call(
        _tc_tail_body,
        grid=(1,),
        in_specs=[pl.BlockSpec((_C, _B, 128), lambda i: (0, 0, _QTAIL // 128))],
        out_specs=[
            pl.BlockSpec((_C, _B, 1), lambda i: (0, 0, 0)),
            pl.BlockSpec((_C, _B, 1), lambda i: (0, 0, 0)),
        ],
        out_shape=[
            jax.ShapeDtypeStruct((_C, _B, 1), jnp.float32),
            jax.ShapeDtypeStruct((_C, _B, 1), jnp.int32),
        ],
    )(logits_t)


def _sc_reduce_body(x_hbm, vals_out, idx_out,
                    buf0, buf1, mstate, istate, vrow, irow, sem0, sem1):
    wid = lax.axis_index("s") * _NC + lax.axis_index("c")
    bufs = (buf0, buf1)
    sems = (sem0, sem1)
    lane = lax.iota(jnp.int32, 16)

    for rep in range(_NU // 32):
        u = wid + 32 * rep
        cc = _CT + u // 2
        tr = u % 2

        def _src(k):
            q0, w = _CHUNKS[k]
            return x_hbm.at[cc, pl.ds(8 * tr, 8), pl.ds(q0, w)]

        def _dst(k):
            w = _CHUNKS[k][1]
            buf = bufs[k % 2]
            return buf if w == 7680 else buf.at[:, pl.ds(0, w)]

        for b_loc in range(8):
            mstate[b_loc] = jnp.full((16,), -1.0, jnp.float32)
            istate[b_loc] = jnp.zeros((16,), jnp.int32)

        handles = [None, None, None]
        handles[0] = pltpu.async_copy(_src(0), _dst(0), sems[0])
        for k in range(3):
            if k + 1 < 3:
                handles[k + 1] = pltpu.async_copy(_src(k + 1), _dst(k + 1),
                                                  sems[(k + 1) % 2])
            handles[k].wait()
            q0, w = _CHUNKS[k]
            buf = bufs[k % 2]
            for b_loc in range(8):

                def _step(i, carry, b_loc=b_loc, buf=buf, q0=q0):
                    m, idx = carry
                    for j in range(4):
                        v = buf[b_loc, pl.ds(i * 64 + j * 16, 16)]
                        p = 1.0 / (1.0 + jnp.exp(-v))
                        qvec = lane + (q0 + i * 64 + j * 16)
                        gt = p > m
                        m = jnp.where(gt, p, m)
                        idx = jnp.where(gt, qvec, idx)
                    return (m, idx)

                m0 = mstate[b_loc]
                i0 = istate[b_loc]
                m, idx = lax.fori_loop(0, w // 64, _step, (m0, i0))
                mstate[b_loc] = m
                istate[b_loc] = idx

        for b_loc in range(8):
            m = mstate[b_loc]
            idx = istate[b_loc]
            gmax = jnp.max(m)
            gidx = jnp.min(jnp.where(m == gmax, idx, _Q))
            vrow[b_loc] = jnp.full((16,), gmax, jnp.float32)
            irow[b_loc] = jnp.full((16,), gidx, jnp.int32)
        pltpu.sync_copy(vrow, vals_out.at[u])
        pltpu.sync_copy(irow, idx_out.at[u])


@functools.cache
def _sc_reduce():
    return functools.partial(
        pl.kernel,
        mesh=plsc.VectorSubcoreMesh(core_axis_name="c", subcore_axis_name="s"),
        compiler_params=pltpu.CompilerParams(use_tc_tiling_on_sc=True,
                                             needs_layout_passes=False),
        out_type=[
            jax.ShapeDtypeStruct((_NU, 8, 16), jnp.float32),
            jax.ShapeDtypeStruct((_NU, 8, 16), jnp.int32),
        ],
        scratch_types=[
            pltpu.VMEM((8, 7680), jnp.float32),
            pltpu.VMEM((8, 7680), jnp.float32),
            pltpu.VMEM((8, 16), jnp.float32),
            pltpu.VMEM((8, 16), jnp.int32),
            pltpu.VMEM((8, 16), jnp.float32),
            pltpu.VMEM((8, 16), jnp.int32),
            pltpu.SemaphoreType.DMA,
            pltpu.SemaphoreType.DMA,
        ],
    )(_sc_reduce_body)


def _sc_gather_body(vals_hbm, idx_hbm, lab_hbm, boxes_hbm,
                    scores_out, boxes_out,
                    lab_v, gidx_v, sc_v, bidx_v, brow_v, sem):
    wid = lax.axis_index("s") * _NC + lax.axis_index("c")
    b = wid // 2
    pltpu.sync_copy(lab_hbm.at[wid], lab_v)
    gidx_v[...] = lab_v[...] + b * _CP
    pltpu.async_copy(vals_hbm.at[gidx_v], sc_v, sem).wait()
    pltpu.sync_copy(sc_v, scores_out.at[wid])
    pltpu.async_copy(idx_hbm.at[gidx_v], bidx_v, sem).wait()
    bidx_v[...] = bidx_v[...] + b * (4 * _Q)
    for k in range(4):
        gidx_v[...] = bidx_v[...] + k * _Q
        pltpu.async_copy(boxes_hbm.at[gidx_v], brow_v.at[k], sem).wait()
    pltpu.sync_copy(brow_v, boxes_out.at[wid])


@functools.cache
def _sc_gather():
    return functools.partial(
        pl.kernel,
        mesh=plsc.VectorSubcoreMesh(core_axis_name="c", subcore_axis_name="s"),
        compiler_params=pltpu.CompilerParams(use_tc_tiling_on_sc=False),
        out_type=[
            jax.ShapeDtypeStruct((_B * 2, 16), jnp.float32),
            jax.ShapeDtypeStruct((_B * 2, 4, 16), jnp.float32),
        ],
        scratch_types=[
            pltpu.VMEM((16,), jnp.int32),
            pltpu.VMEM((16,), jnp.int32),
            pltpu.VMEM((16,), jnp.float32),
            pltpu.VMEM((16,), jnp.int32),
            pltpu.VMEM((4, 16), jnp.float32),
            pltpu.SemaphoreType.DMA,
        ],
    )(_sc_gather_body)


def kernel(pred_logits, pred_boxes, target_sizes, target_labels):
    del target_sizes
    logits_t = pred_logits.transpose(2, 0, 1)
    vals_tc, idx_tc = _tc_reduce(logits_t)
    vals_tl, idx_tl = _tc_tail(logits_t)
    vals_sc, idx_sc = _sc_reduce()(logits_t)
    v_sc = vals_sc[:, :, 0].reshape(_NSC, 16).T
    i_sc = idx_sc[:, :, 0].reshape(_NSC, 16).T
    v_tl = vals_tl[_CT:, :, 0].T
    i_tl = idx_tl[_CT:, :, 0].T
    later = v_tl > v_sc
    v_sc = jnp.where(later, v_tl, v_sc)
    i_sc = jnp.where(later, i_tl, i_sc)
    v_full = jnp.concatenate([vals_tc[:, :, 0].T, v_sc], axis=1)
    i_full = jnp.concatenate([idx_tc[:, :, 0].T, i_sc], axis=1)
    vals = jnp.pad(v_full, ((0, 0), (0, _CP - _C))).reshape(-1)
    idx = jnp.pad(i_full, ((0, 0), (0, _CP - _C))).reshape(-1)
    lab = jnp.pad(target_labels, ((0, 0), (0, _LP - _L))).reshape(_B * 2, 16)
    boxes_kq = pred_boxes.transpose(0, 2, 1).reshape(-1)
    scores32, boxes32 = _sc_gather()(vals, idx, lab, boxes_kq)
    scores = scores32.reshape(_B, _LP)[:, :_L]
    boxes = (boxes32.reshape(_B, 2, 4, 16).transpose(0, 1, 3, 2)
             .reshape(_B, _LP, 4)[:, :_L, :])
    return (scores, target_labels, boxes)


# speedup vs baseline: 1.0118x; 1.0010x over previous
import functools

import jax
import jax.numpy as jnp
from jax import lax
from jax.experimental import pallas as pl
from jax.experimental.pallas import tpu as pltpu
from jax.experimental.pallas import tpu_sc as plsc

_B, _Q, _C, _L = 16, 20000, 91, 20
_CP = 128
_LP = 32
_NC = 2
_CT = 59
_NSC = _C - _CT
_NU = 2 * _NSC
_CHUNKS = ((0, 7680), (7680, 7680), (15360, 4608))
_QTAIL = 19968


def _tc_reduce_body(x_ref, vals_ref, idx_ref):
    p = jax.nn.sigmoid(x_ref[0])
    bm = jnp.max(p, axis=1, keepdims=True)
    qio = lax.broadcasted_iota(jnp.int32, (_B, _Q), 1)
    bidx = jnp.min(jnp.where(p == bm, qio, _Q), axis=1, keepdims=True)
    vals_ref[0] = bm
    idx_ref[0] = bidx


def _tc_reduce(logits_t):
    return pl.pallas_call(
        _tc_reduce_body,
        grid=(_CT,),
        in_specs=[pl.BlockSpec((1, _B, _Q), lambda c: (c, 0, 0))],
        out_specs=[
            pl.BlockSpec((1, _B, 1), lambda c: (c, 0, 0)),
            pl.BlockSpec((1, _B, 1), lambda c: (c, 0, 0)),
        ],
        out_shape=[
            jax.ShapeDtypeStruct((_CT, _B, 1), jnp.float32),
            jax.ShapeDtypeStruct((_CT, _B, 1), jnp.int32),
        ],
    )(logits_t)


def _tc_tail_body(x_ref, vals_ref, idx_ref):
    lanes = lax.broadcasted_iota(jnp.int32, (_C, _B, 128), 2)
    valid = lanes < (_Q - _QTAIL)
    p = jnp.where(valid, jax.nn.sigmoid(x_ref[...]), -1.0)
    bm = jnp.max(p, axis=2, keepdims=True)
    bidx = jnp.min(jnp.where(p == bm, lanes + _QTAIL, _Q),
                   axis=2, keepdims=True)
    vals_ref[...] = bm
    idx_ref[...] = bidx


def _tc_tail(logits_t):
    return pl.pallas_call(
        _tc_tail_body,
        grid=(1,),
        in_specs=[pl.BlockSpec((_C, _B, 128), lambda i: (0, 0, _QTAIL // 128))],
        out_specs=[
            pl.BlockSpec((_C, _B, 1), lambda i: (0, 0, 0)),
            pl.BlockSpec((_C, _B, 1), lambda i: (0, 0, 0)),
        ],
        out_shape=[
            jax.ShapeDtypeStruct((_C, _B, 1), jnp.float32),
            jax.ShapeDtypeStruct((_C, _B, 1), jnp.int32),
        ],
    )(logits_t)


def _sc_reduce_body(x_hbm, vals_out, idx_out,
                    buf0, buf1, mstate, istate, vrow, irow, sem0, sem1):
    wid = lax.axis_index("s") * _NC + lax.axis_index("c")
    bufs = (buf0, buf1)
    sems = (sem0, sem1)
    lane = lax.iota(jnp.int32, 16)

    for rep in range(_NU // 32):
        u = wid + 32 * rep
        cc = _CT + u // 2
        tr = u % 2

        def _src(k):
            q0, w = _CHUNKS[k]
            return x_hbm.at[cc, pl.ds(8 * tr, 8), pl.ds(q0, w)]

        def _dst(k):
            w = _CHUNKS[k][1]
            buf = bufs[k % 2]
            return buf if w == 7680 else buf.at[:, pl.ds(0, w)]

        for b_loc in range(8):
            mstate[b_loc] = jnp.full((16,), -1.0, jnp.float32)
            istate[b_loc] = jnp.zeros((16,), jnp.int32)

        handles = [None, None, None]
        handles[0] = pltpu.async_copy(_src(0), _dst(0), sems[0])
        for k in range(3):
            if k + 1 < 3:
                handles[k + 1] = pltpu.async_copy(_src(k + 1), _dst(k + 1),
                                                  sems[(k + 1) % 2])
            handles[k].wait()
            q0, w = _CHUNKS[k]
            buf = bufs[k % 2]
            for b_loc in range(8):

                def _step(i, carry, b_loc=b_loc, buf=buf, q0=q0):
                    ms, idxs = carry
                    ms, idxs = list(ms), list(idxs)
                    for j in range(4):
                        v = buf[b_loc, pl.ds(i * 64 + j * 16, 16)]
                        p = 1.0 / (1.0 + jnp.exp(-v))
                        qvec = lane + (q0 + i * 64 + j * 16)
                        gt = p > ms[j]
                        ms[j] = jnp.where(gt, p, ms[j])
                        idxs[j] = jnp.where(gt, qvec, idxs[j])
                    return (tuple(ms), tuple(idxs))

                m0 = mstate[b_loc]
                i0 = istate[b_loc]
                ms, idxs = lax.fori_loop(
                    0, w // 64, _step, ((m0,) * 4, (i0,) * 4))
                m = jnp.maximum(jnp.maximum(ms[0], ms[1]),
                                jnp.maximum(ms[2], ms[3]))
                idx = jnp.full((16,), _Q, jnp.int32)
                for j in range(4):
                    idx = jnp.minimum(idx, jnp.where(ms[j] == m, idxs[j], _Q))
                mstate[b_loc] = m
                istate[b_loc] = idx

        for b_loc in range(8):
            m = mstate[b_loc]
            idx = istate[b_loc]
            gmax = jnp.max(m)
            gidx = jnp.min(jnp.where(m == gmax, idx, _Q))
            vrow[b_loc] = jnp.full((16,), gmax, jnp.float32)
            irow[b_loc] = jnp.full((16,), gidx, jnp.int32)
        pltpu.sync_copy(vrow, vals_out.at[u])
        pltpu.sync_copy(irow, idx_out.at[u])


@functools.cache
def _sc_reduce():
    return functools.partial(
        pl.kernel,
        mesh=plsc.VectorSubcoreMesh(core_axis_name="c", subcore_axis_name="s"),
        compiler_params=pltpu.CompilerParams(use_tc_tiling_on_sc=True,
                                             needs_layout_passes=False),
        out_type=[
            jax.ShapeDtypeStruct((_NU, 8, 16), jnp.float32),
            jax.ShapeDtypeStruct((_NU, 8, 16), jnp.int32),
        ],
        scratch_types=[
            pltpu.VMEM((8, 7680), jnp.float32),
            pltpu.VMEM((8, 7680), jnp.float32),
            pltpu.VMEM((8, 16), jnp.float32),
            pltpu.VMEM((8, 16), jnp.int32),
            pltpu.VMEM((8, 16), jnp.float32),
            pltpu.VMEM((8, 16), jnp.int32),
            pltpu.SemaphoreType.DMA,
            pltpu.SemaphoreType.DMA,
        ],
    )(_sc_reduce_body)


def _sc_gather_body(vals_hbm, idx_hbm, lab_hbm, boxes_hbm,
                    scores_out, boxes_out,
                    lab_v, gidx_v, sc_v, bidx_v, brow_v, sem):
    wid = lax.axis_index("s") * _NC + lax.axis_index("c")
    b = wid // 2
    pltpu.sync_copy(lab_hbm.at[wid], lab_v)
    gidx_v[...] = lab_v[...] + b * _CP
    pltpu.async_copy(vals_hbm.at[gidx_v], sc_v, sem).wait()
    pltpu.sync_copy(sc_v, scores_out.at[wid])
    pltpu.async_copy(idx_hbm.at[gidx_v], bidx_v, sem).wait()
    bidx_v[...] = bidx_v[...] + b * (4 * _Q)
    for k in range(4):
        gidx_v[...] = bidx_v[...] + k * _Q
        pltpu.async_copy(boxes_hbm.at[gidx_v], brow_v.at[k], sem).wait()
    pltpu.sync_copy(brow_v, boxes_out.at[wid])


@functools.cache
def _sc_gather():
    return functools.partial(
        pl.kernel,
        mesh=plsc.VectorSubcoreMesh(core_axis_name="c", subcore_axis_name="s"),
        compiler_params=pltpu.CompilerParams(use_tc_tiling_on_sc=False),
        out_type=[
            jax.ShapeDtypeStruct((_B * 2, 16), jnp.float32),
            jax.ShapeDtypeStruct((_B * 2, 4, 16), jnp.float32),
        ],
        scratch_types=[
            pltpu.VMEM((16,), jnp.int32),
            pltpu.VMEM((16,), jnp.int32),
            pltpu.VMEM((16,), jnp.float32),
            pltpu.VMEM((16,), jnp.int32),
            pltpu.VMEM((4, 16), jnp.float32),
            pltpu.SemaphoreType.DMA,
        ],
    )(_sc_gather_body)


def kernel(pred_logits, pred_boxes, target_sizes, target_labels):
    del target_sizes
    logits_t = pred_logits.transpose(2, 0, 1)
    vals_tc, idx_tc = _tc_reduce(logits_t)
    vals_tl, idx_tl = _tc_tail(logits_t)
    vals_sc, idx_sc = _sc_reduce()(logits_t)
    v_sc = vals_sc[:, :, 0].reshape(_NSC, 16).T
    i_sc = idx_sc[:, :, 0].reshape(_NSC, 16).T
    v_tl = vals_tl[_CT:, :, 0].T
    i_tl = idx_tl[_CT:, :, 0].T
    later = v_tl > v_sc
    v_sc = jnp.where(later, v_tl, v_sc)
    i_sc = jnp.where(later, i_tl, i_sc)
    v_full = jnp.concatenate([vals_tc[:, :, 0].T, v_sc], axis=1)
    i_full = jnp.concatenate([idx_tc[:, :, 0].T, i_sc], axis=1)
    vals = jnp.pad(v_full, ((0, 0), (0, _CP - _C))).reshape(-1)
    idx = jnp.pad(i_full, ((0, 0), (0, _CP - _C))).reshape(-1)
    lab = jnp.pad(target_labels, ((0, 0), (0, _LP - _L))).reshape(_B * 2, 16)
    boxes_kq = pred_boxes.transpose(0, 2, 1).reshape(-1)
    scores32, boxes32 = _sc_gather()(vals, idx, lab, boxes_kq)
    scores = scores32.reshape(_B, _LP)[:, :_L]
    boxes = (boxes32.reshape(_B, 2, 4, 16).transpose(0, 1, 3, 2)
             .reshape(_B, _LP, 4)[:, :_L, :])
    return (scores, target_labels, boxes)
